# Initial kernel scaffold; baseline (speedup 1.0000x reference)
import jax
import jax.numpy as jnp
from jax.experimental import pallas as pl


def kernel(logits, monoaction_mask, monofield_base_converter):
    raise NotImplementedError("write your pallas kernel here")


# speedup vs baseline: 1.1400x; 1.1400x over previous
import functools

import jax
import jax.numpy as jnp
from jax.experimental import pallas as pl
from jax.experimental.pallas import tpu as pltpu

GRID = (48, 48)
NCELL = GRID[0] * GRID[1]
NA = 6
NB = 4096
BLK = 512
FMIN = jnp.finfo(jnp.float32).min


def _body(mask_ref, logits_ref, conv_ref, out_ref):
    logits = jnp.broadcast_to(logits_ref[...], (2 * NA, NCELL))
    conv = jnp.broadcast_to(conv_ref[...], (2 * NA, NCELL))
    cls = jax.lax.broadcasted_iota(jnp.int32, (2 * NA, NCELL), 0)
    onehot = (conv == jnp.where(cls >= NA, cls - NA, cls)).astype(jnp.float32)
    wt = jnp.where(cls < NA, logits, 1.0) * onehot

    maskf = mask_ref[...].astype(jnp.float32)
    acc = jax.lax.dot_general(
        maskf, wt, (((1,), (1,)), ((), ())),
        preferred_element_type=jnp.float32)

    sums = acc[:, :NA]
    counts = acc[:, NA:]
    total = jnp.sum(counts, axis=1, keepdims=True)
    col = jax.lax.broadcasted_iota(jnp.int32, sums.shape, 1)
    scaled = jnp.where(col == NA - 1, sums * (1.0 / 225.0), sums)
    out = jnp.where(counts > 0.5, scaled, FMIN)
    out_ref[...] = jnp.where((total < 0.5) & (col == 0), 1.0, out)


def kernel(logits, monoaction_mask, monofield_base_converter):
    mask2d = monoaction_mask.reshape(NB, NCELL)
    grid = NB // BLK
    out = pl.pallas_call(
        _body,
        grid=(grid,),
        in_specs=[
            pl.BlockSpec((BLK, NCELL), lambda i: (i, 0)),
            pl.BlockSpec((1, NCELL), lambda i: (0, 0)),
            pl.BlockSpec((1, NCELL), lambda i: (0, 0)),
        ],
        out_specs=pl.BlockSpec((BLK, NA), lambda i: (i, 0)),
        out_shape=jax.ShapeDtypeStruct((NB, NA), jnp.float32),
    )(mask2d, logits.reshape(1, NCELL),
      monofield_base_converter.reshape(1, NCELL))
    return out


# speedup vs baseline: 2.6227x; 2.3007x over previous
import jax
import jax.numpy as jnp
from jax.experimental import pallas as pl

GRID = (48, 48)
NCELL = GRID[0] * GRID[1]
NA = 6
NB = 4096
BLK = 1024
FMIN = jnp.finfo(jnp.float32).min


def _body(mask_ref, logits_ref, conv_ref, out_ref):
    logits = jnp.broadcast_to(logits_ref[...], (2 * NA, NCELL))
    conv = jnp.broadcast_to(conv_ref[...], (2 * NA, NCELL))
    cls = jax.lax.broadcasted_iota(jnp.int32, (2 * NA, NCELL), 0)
    onehot = (conv == jnp.where(cls >= NA, cls - NA, cls)).astype(jnp.float32)
    wt = jnp.where(cls < NA, logits, 1.0) * onehot

    maskf = mask_ref[...].reshape(NCELL, BLK).astype(jnp.float32)
    acc = jax.lax.dot_general(
        wt, maskf, (((1,), (0,)), ((), ())),
        preferred_element_type=jnp.float32)

    sums = acc[:NA, :]
    counts = acc[NA:, :]
    total = jnp.sum(counts, axis=0, keepdims=True)
    row = jax.lax.broadcasted_iota(jnp.int32, sums.shape, 0)
    scaled = jnp.where(row == NA - 1, sums * (1.0 / 225.0), sums)
    out = jnp.where(counts > 0.5, scaled, FMIN)
    out_ref[...] = jnp.where((total < 0.5) & (row == 0), 1.0, out)


def kernel(logits, monoaction_mask, monofield_base_converter):
    mask_t = monoaction_mask.transpose(1, 2, 0)
    grid = NB // BLK
    out_t = pl.pallas_call(
        _body,
        grid=(grid,),
        in_specs=[
            pl.BlockSpec((GRID[0], GRID[1], BLK), lambda i: (0, 0, i)),
            pl.BlockSpec((1, NCELL), lambda i: (0, 0)),
            pl.BlockSpec((1, NCELL), lambda i: (0, 0)),
        ],
        out_specs=pl.BlockSpec((NA, BLK), lambda i: (0, i)),
        out_shape=jax.ShapeDtypeStruct((NA, NB), jnp.float32),
    )(mask_t, logits.reshape(1, NCELL),
      monofield_base_converter.reshape(1, NCELL))
    return out_t.T
